# baseline (speedup 1.0000x reference)
import functools

import jax
import jax.numpy as jnp
from jax import lax
from jax.experimental import pallas as pl
from jax.experimental.pallas import tpu as pltpu
from jax.experimental.pallas import tpu_sc as plsc

_N_FIELDS = 26
_N_CONT = 13
_VOCAB = 100000
_EMB_DIM = 16
_BATCH = 16384
_EPS = 1e-5

_NC = 2
_NS = 16
_NW = _NC * _NS
_TOTAL_ROWS = _BATCH * _N_FIELDS
_ROWS_PER_W = _TOTAL_ROWS // _NW
_N_CHUNK = 8
_CH = _ROWS_PER_W // _N_CHUNK


def _prep_body(x_ref, g_ref, b_ref, idx_ref, cont_ref):
    x = x_ref[...]
    ids = x[:, :_N_FIELDS].astype(jnp.int32)
    offs = lax.broadcasted_iota(jnp.int32, (1, _N_FIELDS), 1) * (_VOCAB + 1)
    idx_ref[...] = ids + offs
    xc = x[:, _N_FIELDS:]
    mean = jnp.mean(xc, axis=0, keepdims=True)
    d = xc - mean
    var = jnp.mean(d * d, axis=0, keepdims=True)
    cont_ref[...] = d * lax.rsqrt(var + _EPS) * g_ref[...] + b_ref[...]


_prep = pl.pallas_call(
    _prep_body,
    out_shape=(
        jax.ShapeDtypeStruct((_BATCH, _N_FIELDS), jnp.int32),
        jax.ShapeDtypeStruct((_BATCH, _N_CONT), jnp.float32),
    ),
)


_sc_mesh = plsc.VectorSubcoreMesh(core_axis_name="c", subcore_axis_name="s")


@functools.partial(
    pl.kernel,
    mesh=_sc_mesh,
    out_type=jax.ShapeDtypeStruct((_TOTAL_ROWS, _EMB_DIM), jnp.float32),
    compiler_params=pltpu.CompilerParams(use_tc_tiling_on_sc=False),
    scratch_types=[
        pltpu.VMEM((_CH,), jnp.int32),
        pltpu.VMEM((_CH, _EMB_DIM), jnp.float32),
        pltpu.SemaphoreType.DMA,
    ],
)
def _gather(table_hbm, idx_hbm, out_hbm, idx_v, rows_v, gsem):
    wid = lax.axis_index("s") * _NC + lax.axis_index("c")
    base = wid * _ROWS_PER_W
    for j in range(_N_CHUNK):
        pltpu.sync_copy(idx_hbm.at[wid, j], idx_v)
        pltpu.async_copy(table_hbm.at[idx_v], rows_v, gsem).wait()
        pltpu.sync_copy(rows_v, out_hbm.at[pl.ds(base + j * _CH, _CH)])


def kernel(X, W_emb, gamma, beta):
    idx2d, x_cont = _prep(X, gamma.reshape(1, _N_CONT), beta.reshape(1, _N_CONT))
    idx = idx2d.reshape(_NW, _N_CHUNK, _CH)
    table = W_emb.reshape((_N_FIELDS * (_VOCAB + 1), _EMB_DIM))
    rows = _gather(table, idx)
    x_emb = rows.reshape(_BATCH, _N_FIELDS * _EMB_DIM)
    return (x_emb, x_cont)


# speedup vs baseline: 1.7848x; 1.7848x over previous
import functools

import jax
import jax.numpy as jnp
from jax import lax
from jax.experimental import pallas as pl
from jax.experimental.pallas import tpu as pltpu
from jax.experimental.pallas import tpu_sc as plsc

_N_FIELDS = 26
_N_CONT = 13
_VOCAB = 100000
_EMB_DIM = 16
_BATCH = 16384
_EPS = 1e-5

_NC = 2
_NS = 16
_NW = _NC * _NS
_TOTAL_ROWS = _BATCH * _N_FIELDS
_ROWS_PER_W = _TOTAL_ROWS // _NW
_N_CHUNK = 8
_CH = _ROWS_PER_W // _N_CHUNK


def _prep_body(x_ref, g_ref, b_ref, idx_ref, cont_ref):
    x = x_ref[...]
    ids = x[:, :_N_FIELDS].astype(jnp.int32)
    offs = lax.broadcasted_iota(jnp.int32, (1, _N_FIELDS), 1) * (_VOCAB + 1)
    idx_ref[...] = ids + offs
    xc = x[:, _N_FIELDS:]
    mean = jnp.mean(xc, axis=0, keepdims=True)
    d = xc - mean
    var = jnp.mean(d * d, axis=0, keepdims=True)
    cont_ref[...] = d * lax.rsqrt(var + _EPS) * g_ref[...] + b_ref[...]


_prep = pl.pallas_call(
    _prep_body,
    out_shape=(
        jax.ShapeDtypeStruct((_BATCH, _N_FIELDS), jnp.int32),
        jax.ShapeDtypeStruct((_BATCH, _N_CONT), jnp.float32),
    ),
)


_sc_mesh = plsc.VectorSubcoreMesh(core_axis_name="c", subcore_axis_name="s")


@functools.partial(
    pl.kernel,
    mesh=_sc_mesh,
    out_type=jax.ShapeDtypeStruct((_TOTAL_ROWS, _EMB_DIM), jnp.float32),
    compiler_params=pltpu.CompilerParams(use_tc_tiling_on_sc=False),
    scratch_types=[
        pltpu.VMEM((_CH,), jnp.int32),
        pltpu.VMEM((_CH, _EMB_DIM), jnp.float32),
        pltpu.SemaphoreType.DMA,
    ],
)
def _gather(table_hbm, idx_hbm, out_hbm, idx_v, rows_v, gsem):
    wid = lax.axis_index("s") * _NC + lax.axis_index("c")
    base = wid * _ROWS_PER_W
    for j in range(_N_CHUNK):
        pltpu.sync_copy(idx_hbm.at[wid, j], idx_v)
        pltpu.async_copy(table_hbm.at[idx_v], rows_v, gsem).wait()
        pltpu.sync_copy(rows_v, out_hbm.at[pl.ds(base + j * _CH, _CH)])


_VB = 8192
_NVB = -(-(_VOCAB + 1) // _VB)


def _transpose_body(w_ref, o_ref):
    o_ref[...] = jnp.transpose(w_ref[...], (0, 2, 1))


_transpose = pl.pallas_call(
    _transpose_body,
    grid=(_N_FIELDS, _NVB),
    in_specs=[pl.BlockSpec((1, _EMB_DIM, _VB), lambda f, j: (f, 0, j))],
    out_specs=pl.BlockSpec((1, _VB, _EMB_DIM), lambda f, j: (f, j, 0)),
    out_shape=jax.ShapeDtypeStruct((_N_FIELDS, _VOCAB + 1, _EMB_DIM), jnp.float32),
)


def kernel(X, W_emb, gamma, beta):
    idx2d, x_cont = _prep(X, gamma.reshape(1, _N_CONT), beta.reshape(1, _N_CONT))
    idx = idx2d.reshape(_NW, _N_CHUNK, _CH)
    w_vm = jnp.transpose(W_emb, (0, 2, 1))
    w_rm = _transpose(w_vm)
    table = w_rm.reshape((_N_FIELDS * (_VOCAB + 1), _EMB_DIM))
    rows = _gather(table, idx)
    x_emb = rows.reshape(_BATCH, _N_FIELDS * _EMB_DIM)
    return (x_emb, x_cont)


# speedup vs baseline: 2.7073x; 1.5169x over previous
import functools

import jax
import jax.numpy as jnp
from jax import lax
from jax.experimental import pallas as pl
from jax.experimental.pallas import tpu as pltpu
from jax.experimental.pallas import tpu_sc as plsc

_N_FIELDS = 26
_N_CONT = 13
_VOCAB = 100000
_EMB_DIM = 16
_BATCH = 16384
_EPS = 1e-5

_NC = 2
_NS = 16
_NW = _NC * _NS
_TOTAL_ROWS = _BATCH * _N_FIELDS
_ROWS_PER_W = _TOTAL_ROWS // _NW
_N_CHUNK = 8
_CH = _ROWS_PER_W // _N_CHUNK
_VPAD = 100096
_TOTAL_ROWS_T = _N_FIELDS * _VPAD
_N_DB = 13 * 17
_DB = _VPAD // (16 * 17)


def _prep_body(x_ref, g_ref, b_ref, idx_ref, cont_ref):
    x = x_ref[...]
    ids = x[:, :_N_FIELDS].astype(jnp.int32)
    offs = lax.broadcasted_iota(jnp.int32, (1, _N_FIELDS), 1) * _VPAD
    idx_ref[...] = ids + offs
    xc = x[:, _N_FIELDS:]
    mean = jnp.mean(xc, axis=0, keepdims=True)
    d = xc - mean
    var = jnp.mean(d * d, axis=0, keepdims=True)
    cont_ref[...] = d * lax.rsqrt(var + _EPS) * g_ref[...] + b_ref[...]


_prep = pl.pallas_call(
    _prep_body,
    out_shape=(
        jax.ShapeDtypeStruct((_BATCH, _N_FIELDS), jnp.int32),
        jax.ShapeDtypeStruct((_BATCH, _N_CONT), jnp.float32),
    ),
)


_sc_mesh = plsc.VectorSubcoreMesh(core_axis_name="c", subcore_axis_name="s")


@functools.partial(
    pl.kernel,
    mesh=_sc_mesh,
    out_type=jax.ShapeDtypeStruct((_TOTAL_ROWS, _EMB_DIM), jnp.float32),
    compiler_params=pltpu.CompilerParams(use_tc_tiling_on_sc=False),
    scratch_types=[
        pltpu.VMEM((_CH,), jnp.int32),
        pltpu.VMEM((_CH, _EMB_DIM), jnp.float32),
        pltpu.SemaphoreType.DMA,
    ],
)
def _gather(table_hbm, idx_hbm, out_hbm, idx_v, rows_v, gsem):
    wid = lax.axis_index("s") * _NC + lax.axis_index("c")
    base = wid * _ROWS_PER_W
    for j in range(_N_CHUNK):
        pltpu.sync_copy(idx_hbm.at[wid, j], idx_v)
        pltpu.async_copy(table_hbm.at[idx_v], rows_v, gsem).wait()
        pltpu.sync_copy(rows_v, out_hbm.at[pl.ds(base + j * _CH, _CH)])


_VB = 8192
_NVB = -(-_VPAD // _VB)


def _transpose_body(w_ref, o_ref):
    o_ref[...] = jnp.transpose(w_ref[...], (0, 2, 1))


_transpose = pl.pallas_call(
    _transpose_body,
    grid=(_N_FIELDS, _NVB),
    in_specs=[pl.BlockSpec((1, _EMB_DIM, _VB), lambda f, j: (f, 0, j))],
    out_specs=pl.BlockSpec((1, _VB, _EMB_DIM), lambda f, j: (f, j, 0)),
    out_shape=jax.ShapeDtypeStruct((_N_FIELDS, _VPAD, _EMB_DIM), jnp.float32),
)


@functools.partial(
    pl.kernel,
    mesh=_sc_mesh,
    out_type=jax.ShapeDtypeStruct((_TOTAL_ROWS_T, _EMB_DIM), jnp.float32),
    scratch_types=[
        pltpu.VMEM((_DB, _EMB_DIM), jnp.float32),
    ],
)
def _detile(src_hbm, dst_hbm, buf_v):
    wid = lax.axis_index("s") * _NC + lax.axis_index("c")
    base = wid * _N_DB * _DB
    for j in range(_N_DB):
        pltpu.sync_copy(src_hbm.at[pl.ds(base + j * _DB, _DB)], buf_v)
        pltpu.sync_copy(buf_v, dst_hbm.at[pl.ds(base + j * _DB, _DB)])


def kernel(X, W_emb, gamma, beta):
    idx2d, x_cont = _prep(X, gamma.reshape(1, _N_CONT), beta.reshape(1, _N_CONT))
    idx = idx2d.reshape(_NW, _N_CHUNK, _CH)
    w_vm = jnp.transpose(W_emb, (0, 2, 1))
    w_rm = _transpose(w_vm)
    table_t = w_rm.reshape((_TOTAL_ROWS_T, _EMB_DIM))
    table = _detile(table_t)
    rows = _gather(table, idx)
    x_emb = rows.reshape(_BATCH, _N_FIELDS * _EMB_DIM)
    return (x_emb, x_cont)


# speedup vs baseline: 6.6484x; 2.4557x over previous
import functools

import jax
import jax.numpy as jnp
from jax import lax
from jax.experimental import pallas as pl
from jax.experimental.pallas import tpu as pltpu
from jax.experimental.pallas import tpu_sc as plsc

_N_FIELDS = 26
_N_CONT = 13
_VOCAB = 100000
_EMB_DIM = 16
_BATCH = 16384
_EPS = 1e-5

_NC = 2
_NS = 16
_NW = _NC * _NS
_TOTAL_ROWS = _BATCH * _N_FIELDS
_ROWS_PER_W = _TOTAL_ROWS // _NW
_N_CHUNK = 8
_CH = _ROWS_PER_W // _N_CHUNK
_VPAD = 100352
_TOTAL_ROWS_T = _N_FIELDS * _VPAD
_W128 = _N_FIELDS * _VPAD // 8
_N_DB = 98
_DB = _W128 // _NW // _N_DB


def _prep_body(x_ref, g_ref, b_ref, idx_ref, cont_ref):
    x = x_ref[...]
    ids = x[:, :_N_FIELDS].astype(jnp.int32)
    offs = lax.broadcasted_iota(jnp.int32, (1, _N_FIELDS), 1) * _VPAD
    idx_ref[...] = ids + offs
    xc = x[:, _N_FIELDS:]
    mean = jnp.mean(xc, axis=0, keepdims=True)
    d = xc - mean
    var = jnp.mean(d * d, axis=0, keepdims=True)
    cont_ref[...] = d * lax.rsqrt(var + _EPS) * g_ref[...] + b_ref[...]


_prep = pl.pallas_call(
    _prep_body,
    out_shape=(
        jax.ShapeDtypeStruct((_BATCH, _N_FIELDS), jnp.int32),
        jax.ShapeDtypeStruct((_BATCH, _N_CONT), jnp.float32),
    ),
)


_sc_mesh = plsc.VectorSubcoreMesh(core_axis_name="c", subcore_axis_name="s")


@functools.partial(
    pl.kernel,
    mesh=_sc_mesh,
    out_type=jax.ShapeDtypeStruct((_TOTAL_ROWS, _EMB_DIM), jnp.float32),
    compiler_params=pltpu.CompilerParams(use_tc_tiling_on_sc=False),
    scratch_types=[
        pltpu.VMEM((_CH,), jnp.int32),
        pltpu.VMEM((_CH, _EMB_DIM), jnp.float32),
        pltpu.SemaphoreType.DMA,
    ],
)
def _gather(table_hbm, idx_hbm, out_hbm, idx_v, rows_v, gsem):
    wid = lax.axis_index("s") * _NC + lax.axis_index("c")
    base = wid * _ROWS_PER_W
    for j in range(_N_CHUNK):
        pltpu.sync_copy(idx_hbm.at[wid, j], idx_v)
        pltpu.async_copy(table_hbm.at[idx_v], rows_v, gsem).wait()
        pltpu.sync_copy(rows_v, out_hbm.at[pl.ds(base + j * _CH, _CH)])


_VB = 8192
_NVB = -(-_VPAD // _VB)


def _transpose_body(w_ref, o_ref):
    t = jnp.transpose(w_ref[...], (0, 2, 1))
    t4 = t.reshape(1, _VB // 8, 8, _EMB_DIM)
    for m in range(8):
        o_ref[:, :, m * 16:(m + 1) * 16] = t4[:, :, m, :]


_transpose = pl.pallas_call(
    _transpose_body,
    grid=(_N_FIELDS, _NVB),
    in_specs=[pl.BlockSpec((1, _EMB_DIM, _VB), lambda f, j: (f, 0, j))],
    out_specs=pl.BlockSpec((1, _VB // 8, 128), lambda f, j: (f, j, 0)),
    out_shape=jax.ShapeDtypeStruct((_N_FIELDS, _VPAD // 8, 128), jnp.float32),
)


@functools.partial(
    pl.kernel,
    mesh=_sc_mesh,
    out_type=jax.ShapeDtypeStruct((_W128, 128), jnp.float32),
    scratch_types=[
        pltpu.VMEM((_DB, 128), jnp.float32),
    ],
)
def _detile(src_hbm, dst_hbm, buf_v):
    wid = lax.axis_index("s") * _NC + lax.axis_index("c")
    base = wid * _N_DB * _DB
    for j in range(_N_DB):
        pltpu.sync_copy(src_hbm.at[pl.ds(base + j * _DB, _DB)], buf_v)
        pltpu.sync_copy(buf_v, dst_hbm.at[pl.ds(base + j * _DB, _DB)])


def kernel(X, W_emb, gamma, beta):
    idx2d, x_cont = _prep(X, gamma.reshape(1, _N_CONT), beta.reshape(1, _N_CONT))
    idx = idx2d.reshape(_NW, _N_CHUNK, _CH)
    w_vm = jnp.transpose(W_emb, (0, 2, 1))
    w_rm = _transpose(w_vm)
    table128 = _detile(w_rm.reshape(_W128, 128))
    table = table128.reshape(_TOTAL_ROWS_T, _EMB_DIM)
    rows = _gather(table, idx)
    x_emb = rows.reshape(_BATCH, _N_FIELDS * _EMB_DIM)
    return (x_emb, x_cont)
